# baseline (speedup 1.0000x reference)
import functools

import jax
import jax.numpy as jnp
from jax import lax
from jax.experimental import pallas as pl
from jax.experimental.pallas import tpu as pltpu
from jax.experimental.pallas import tpu_sc as plsc

NUM_CLASS = 1000
ALPHA = 0.25
GAMMA = 2.0
EPS = 1e-10

ROWS = 32768
PAIRS = ROWS // 2
BLOCK_R = 2048
N_BLOCKS = PAIRS // BLOCK_R

NC = 2
NS = 16
NW = NC * NS
P_PER_W = PAIRS // NW
CH = 16
N_CH = P_PER_W // CH


def _sim_sc_kernel(desc_hbm, out_hbm, a_v, p_v, o_v):
    wid = lax.axis_index("s") * NC + lax.axis_index("c")
    base = wid * P_PER_W

    @pl.loop(0, N_CH)
    def _chunk(ci):
        row0 = base + ci * CH
        pltpu.sync_copy(desc_hbm.at[pl.ds(row0, CH)], a_v)
        pltpu.sync_copy(desc_hbm.at[pl.ds(PAIRS + row0, CH)], p_v)
        for r in range(CH):
            acc = a_v[r, pl.ds(0, 16)] * p_v[r, pl.ds(0, 16)]
            for k in range(1, 8):
                acc = acc + a_v[r, pl.ds(16 * k, 16)] * p_v[r, pl.ds(16 * k, 16)]
            o_v[r, :] = acc
        pltpu.sync_copy(o_v, out_hbm.at[pl.ds(row0, CH)])


@functools.cache
def _sim_sc():
    return pl.kernel(
        _sim_sc_kernel,
        out_type=jax.ShapeDtypeStruct((PAIRS, 16), jnp.float32),
        mesh=plsc.VectorSubcoreMesh(
            core_axis_name="c", subcore_axis_name="s",
            num_cores=NC, num_subcores=NS),
        scratch_types=[
            pltpu.VMEM((CH, 128), jnp.float32),
            pltpu.VMEM((CH, 128), jnp.float32),
            pltpu.VMEM((CH, 16), jnp.float32),
        ],
    )


def _logpt(x, t):
    row_max = jnp.max(x, axis=1, keepdims=True)
    sumexp = jnp.sum(jnp.exp(x - row_max), axis=1, keepdims=True)
    cols = jax.lax.broadcasted_iota(jnp.int32, x.shape, 1)
    xt = jnp.sum(jnp.where(cols == t, x, 0.0), axis=1, keepdims=True)
    pt = jnp.exp(xt - row_max) / sumexp
    return jnp.log(pt + EPS)


def _lp_kernel(xlo_ref, xhi_ref, tlo_ref, thi_ref, out_ref):
    out_ref[...] = (_logpt(xlo_ref[...], tlo_ref[...])
                    + _logpt(xhi_ref[...], thi_ref[...]))


def _combine_kernel(sim_ref, lp_ref, out_ref):
    sim = jnp.sum(sim_ref[...], axis=1, keepdims=True)
    omp = 1.0 - sim
    out_ref[...] = jnp.sum(-ALPHA * omp * omp * lp_ref[...]).reshape(1, 1)


@jax.jit
def kernel(descriptors, input, target):
    sim16 = _sim_sc()(descriptors)

    tgt2d = target.reshape(ROWS, 1)
    lp = pl.pallas_call(
        _lp_kernel,
        grid=(N_BLOCKS,),
        in_specs=[
            pl.BlockSpec((BLOCK_R, NUM_CLASS), lambda i: (i, 0)),
            pl.BlockSpec((BLOCK_R, NUM_CLASS), lambda i: (i + N_BLOCKS, 0)),
            pl.BlockSpec((BLOCK_R, 1), lambda i: (i, 0)),
            pl.BlockSpec((BLOCK_R, 1), lambda i: (i + N_BLOCKS, 0)),
        ],
        out_specs=pl.BlockSpec((BLOCK_R, 1), lambda i: (i, 0)),
        out_shape=jax.ShapeDtypeStruct((PAIRS, 1), jnp.float32),
        compiler_params=pltpu.CompilerParams(
            dimension_semantics=("parallel",)),
    )(input, input, tgt2d, tgt2d)

    total = pl.pallas_call(
        _combine_kernel,
        out_shape=jax.ShapeDtypeStruct((1, 1), jnp.float32),
    )(sim16, lp)
    return total[0, 0] / ROWS
